# baseline (speedup 1.0000x reference)
import functools

import jax
import jax.numpy as jnp
from jax.experimental import pallas as pl
from jax.experimental.pallas import tpu as pltpu

_MiB = 1024 * 1024


def _pipelined_body(x_hbm, w_vmem, b_ref, o_hbm,
                    x_vmem, y_ref, rd_sems, out_sems,
                    *, schedule, rd_sched, d, n_slots, slot_rows):
    def rd_copy(i):
        c0, cn = rd_sched[i]
        sl = pl.ds(c0, cn)
        return pltpu.make_async_copy(x_hbm.at[sl, :], x_vmem.at[sl, :],
                                     rd_sems.at[i])

    for i in range(len(rd_sched)):
        rd_copy(i).start()

    rd_done = 0
    started = []
    for j, (r0, nr) in enumerate(schedule):
        while rd_done < len(rd_sched) and rd_sched[rd_done][0] < r0 + nr:
            rd_copy(rd_done).wait()
            rd_done += 1
        if j >= n_slots:
            started[j - n_slots].wait()
        slot = j % n_slots
        xt = x_vmem[pl.ds(r0, nr), :]
        y = jax.lax.dot_general(xt, w_vmem[...],
                                dimension_numbers=(((1,), (1,)), ((), ())),
                                preferred_element_type=jnp.float32)
        y = y + b_ref[...]
        y_ref[slot, pl.ds(0, nr), pl.ds(0, d)] = y
        y_ref[slot, pl.ds(0, nr), pl.ds(d, d)] = y
        desc = pltpu.make_async_copy(y_ref.at[slot, pl.ds(0, nr)],
                                     o_hbm.at[pl.ds(r0, nr), :],
                                     out_sems.at[slot])
        desc.start()
        started.append(desc)

    for desc in started[-n_slots:]:
        desc.wait()


def kernel(x, weight, bias):
    B, S, D = x.shape
    rows = B * S
    scale = 2

    rd_chunk = 8
    for cand in (512, 256, 128, 64, 32, 16, 8):
        if rows % cand == 0:
            rd_chunk = cand
            break
    rd_sched = []
    c0 = 0
    while c0 < rows:
        cn = rd_chunk if c0 == 0 else min(4 * rd_chunk, rows - c0)
        rd_sched.append((c0, cn))
        c0 += cn
    n_rd = len(rd_sched)

    tile_rows = 256 if rows % 256 == 0 else rd_chunk
    schedule = []
    r0 = 0
    if rows % 512 == 0 and rows >= 1024:
        while r0 < 512:
            schedule.append((r0, 128))
            r0 += 128
    while r0 < rows:
        nr = min(tile_rows, rows - r0)
        schedule.append((r0, nr))
        r0 += nr
    slot_rows = max(nr for _, nr in schedule)
    n_slots = min(8, len(schedule))

    x2d = x.reshape(rows, D)
    b2d = bias.astype(jnp.float32).reshape(1, D)

    body = functools.partial(_pipelined_body, schedule=tuple(schedule),
                             rd_sched=tuple(rd_sched), d=D,
                             n_slots=n_slots, slot_rows=slot_rows)
    out2d = pl.pallas_call(
        body,
        out_shape=jax.ShapeDtypeStruct((rows, scale * D), x.dtype),
        in_specs=[
            pl.BlockSpec(memory_space=pl.ANY),
            pl.BlockSpec(memory_space=pltpu.VMEM),
            pl.BlockSpec(memory_space=pltpu.VMEM),
        ],
        out_specs=pl.BlockSpec(memory_space=pl.ANY),
        scratch_shapes=[
            pltpu.VMEM((rows, D), jnp.float32),
            pltpu.VMEM((n_slots, slot_rows, scale * D), jnp.float32),
            pltpu.SemaphoreType.DMA((n_rd,)),
            pltpu.SemaphoreType.DMA((n_slots,)),
        ],
        compiler_params=pltpu.CompilerParams(
            vmem_limit_bytes=56 * _MiB,
        ),
    )(x2d, weight, b2d)

    return out2d.reshape(rows, scale, D).reshape(B, S * scale, D)


# speedup vs baseline: 1.0383x; 1.0383x over previous
import functools

import jax
import jax.numpy as jnp
from jax.experimental import pallas as pl
from jax.experimental.pallas import tpu as pltpu

_MiB = 1024 * 1024


def _pipelined_body(x_hbm, w_vmem, b_ref, o_hbm,
                    x_vmem, y_ref, rd_sems, out_sems,
                    *, schedule, rd_sched, d, n_slots, slot_rows):
    def rd_copy(i):
        c0, cn = rd_sched[i]
        sl = pl.ds(c0, cn)
        return pltpu.make_async_copy(x_hbm.at[sl, :], x_vmem.at[sl, :],
                                     rd_sems.at[i])

    for i in range(len(rd_sched)):
        rd_copy(i).start()

    rd_done = 0
    started = []
    for j, (r0, nr) in enumerate(schedule):
        while rd_done < len(rd_sched) and rd_sched[rd_done][0] < r0 + nr:
            rd_copy(rd_done).wait()
            rd_done += 1
        if j >= n_slots:
            started[j - n_slots].wait()
        slot = j % n_slots
        xt = x_vmem[pl.ds(r0, nr), :]
        y = jax.lax.dot_general(xt, w_vmem[...],
                                dimension_numbers=(((1,), (1,)), ((), ())),
                                preferred_element_type=jnp.float32)
        y = y + b_ref[...]
        y_ref[slot, pl.ds(0, nr), pl.ds(0, d)] = y
        y_ref[slot, pl.ds(0, nr), pl.ds(d, d)] = y
        desc = pltpu.make_async_copy(y_ref.at[slot, pl.ds(0, nr)],
                                     o_hbm.at[pl.ds(r0, nr), :],
                                     out_sems.at[slot])
        desc.start()
        started.append(desc)

    for desc in started[-n_slots:]:
        desc.wait()


def kernel(x, weight, bias):
    B, S, D = x.shape
    rows = B * S
    scale = 2

    rd_chunk = 8
    for cand in (512, 256, 128, 64, 32, 16, 8):
        if rows % cand == 0:
            rd_chunk = cand
            break
    rd_sched = []
    c0 = 0
    cn = rd_chunk
    while c0 < rows:
        cn = min(cn, rows - c0)
        rd_sched.append((c0, cn))
        c0 += cn
        cn = min(2 * cn, 8 * rd_chunk)
    n_rd = len(rd_sched)

    schedule = []
    r0 = 0
    if rows % 512 == 0 and rows >= 1024:
        while r0 < 512:
            schedule.append((r0, 128))
            r0 += 128
    while r0 < rows:
        nr = min(rd_chunk, rows - r0)
        schedule.append((r0, nr))
        r0 += nr
    slot_rows = max(nr for _, nr in schedule)
    n_slots = min(5, len(schedule))

    x2d = x.reshape(rows, D)
    b2d = bias.astype(jnp.float32).reshape(1, D)

    body = functools.partial(_pipelined_body, schedule=tuple(schedule),
                             rd_sched=tuple(rd_sched), d=D,
                             n_slots=n_slots, slot_rows=slot_rows)
    out2d = pl.pallas_call(
        body,
        out_shape=jax.ShapeDtypeStruct((rows, scale * D), x.dtype),
        in_specs=[
            pl.BlockSpec(memory_space=pl.ANY),
            pl.BlockSpec(memory_space=pltpu.VMEM),
            pl.BlockSpec(memory_space=pltpu.VMEM),
        ],
        out_specs=pl.BlockSpec(memory_space=pl.ANY),
        scratch_shapes=[
            pltpu.VMEM((rows, D), jnp.float32),
            pltpu.VMEM((n_slots, slot_rows, scale * D), jnp.float32),
            pltpu.SemaphoreType.DMA((n_rd,)),
            pltpu.SemaphoreType.DMA((n_slots,)),
        ],
        compiler_params=pltpu.CompilerParams(
            vmem_limit_bytes=57 * _MiB,
        ),
    )(x2d, weight, b2d)

    return out2d.reshape(rows, scale, D).reshape(B, S * scale, D)
